# baseline (speedup 1.0000x reference)
import functools

import jax
import jax.numpy as jnp
from jax import lax
from jax.experimental import pallas as pl
from jax.experimental.pallas import tpu as pltpu
from jax.experimental.pallas import tpu_sc as plsc

K_SPARSE = 64
_NC, _NS, _NL = 2, 16, 16


def _sc_threshold_body(lat_hbm, out_hbm, row0_v, row1_v, cand_v, thr_v,
                       sem0, sem1, *, nrows, l, k):
    wid = lax.axis_index("s") * _NC + lax.axis_index("c")
    base = wid * nrows
    lanes = lax.iota(jnp.int32, _NL)
    nview = l // 128
    unroll = 8
    hi_key = jnp.full((_NL,), jnp.int32(0x7F800001))

    def process(row_v, rbase, r, acc):
        def cmax_step(i, carry):
            out = list(carry)
            for i2 in range(2):
                for j in range(8):
                    x = row_v[pl.ds(rbase + (i * 2 + i2) * 128 + j * _NL,
                                    _NL)]
                    out[j] = jnp.maximum(out[j], x)
            return tuple(out)

        zero = jnp.zeros((_NL,), jnp.float32)
        M = lax.fori_loop(0, nview // 2, cmax_step, (zero,) * 8)

        def psearch(_, lh):
            lo, hi = lh
            mid = lo + lax.shift_right_logical(hi - lo, 1)
            midf = lax.bitcast_convert_type(mid, jnp.float32)
            cnt = jnp.zeros((_NL,), jnp.int32)
            for j in range(8):
                cnt = cnt + plsc.all_reduce_population_count(M[j] >= midf)
            take = cnt >= k
            return (jnp.where(take, mid, lo), jnp.where(take, hi, mid))

        lo0 = jnp.zeros((_NL,), jnp.int32)
        pivk, _ = lax.fori_loop(0, 18, psearch, (lo0, hi_key))
        pivf = lax.bitcast_convert_type(pivk, jnp.float32)

        def comp_step(i, offs):
            xs, ms, ds = [], [], []
            for j in range(unroll):
                x = row_v[pl.ds(rbase + (i * unroll + j) * _NL, _NL)]
                m = x >= pivf
                xs.append(x)
                ms.append(m)
                ds.append(jnp.where(m, _NL, 0))
            for j in range(unroll):
                plsc.store_scatter(cand_v, [offs], xs[j], mask=ms[j])
                offs = offs + ds[j]
            return offs

        offs = lax.fori_loop(0, l // (_NL * unroll), comp_step, lanes)
        offv = lax.shift_right_logical(offs - lanes, 4)
        nsteps = jnp.max(offv)

        def csearch(_, lh):
            lo, hi = lh
            mid = lo + lax.shift_right_logical(hi - lo, 1)
            midf = lax.bitcast_convert_type(mid, jnp.float32)

            def cnt_step(s2, cv):
                for j in range(2):
                    s = s2 * 2 + j
                    xc = cand_v[pl.ds(s * _NL, _NL)]
                    msk = (xc >= midf) & (offv > s)
                    cv = cv + plsc.all_reduce_population_count(msk)
                return cv

            cv = lax.fori_loop(0, (nsteps + 1) >> 1, cnt_step,
                               jnp.zeros((_NL,), jnp.int32))
            take = cv >= k
            return (jnp.where(take, mid, lo), jnp.where(take, hi, mid))

        tk, _ = lax.fori_loop(0, 31, csearch, (pivk, hi_key))
        tf = lax.bitcast_convert_type(tk, jnp.float32)

        acc = jnp.where(lanes == (r & (_NL - 1)), tf, acc)

        @pl.when((r & (_NL - 1)) == (_NL - 1))
        def _():
            thr_v[pl.ds((r >> 4) * _NL, _NL)] = acc

        return acc

    def copy_half(r, row_v, half, sem):
        return pltpu.make_async_copy(
            lat_hbm.at[base + r], row_v.at[pl.ds(half * l, l)], sem)

    def start_pair(r, row_v, sem):
        copy_half(r, row_v, 0, sem).start()
        copy_half(r + 1, row_v, 1, sem).start()

    def wait_pair(r, row_v, sem):
        copy_half(r, row_v, 0, sem).wait()
        copy_half(r + 1, row_v, 1, sem).wait()

    start_pair(0, row0_v, sem0)

    def quad_step(q, acc):
        r = q * 4
        wait_pair(r, row0_v, sem0)
        start_pair(r + 2, row1_v, sem1)
        acc = process(row0_v, 0, r, acc)
        acc = process(row0_v, l, r + 1, acc)
        wait_pair(r + 2, row1_v, sem1)

        @pl.when(r + 4 < nrows)
        def _():
            start_pair(r + 4, row0_v, sem0)

        acc = process(row1_v, 0, r + 2, acc)
        return process(row1_v, l, r + 3, acc)

    lax.fori_loop(0, nrows // 4, quad_step, jnp.zeros((_NL,), jnp.float32))
    pltpu.sync_copy(thr_v, out_hbm.at[pl.ds(base, nrows)])


def _thresholds_sc(latents, k):
    m, l = latents.shape
    nrows = m // (_NC * _NS)
    import functools as _ft
    f = pl.kernel(
        _ft.partial(_sc_threshold_body, nrows=nrows, l=l, k=k),
        out_type=jax.ShapeDtypeStruct((m,), jnp.float32),
        mesh=plsc.VectorSubcoreMesh(core_axis_name="c", subcore_axis_name="s"),
        compiler_params=pltpu.CompilerParams(needs_layout_passes=False),
        scratch_types=[
            pltpu.VMEM((2 * l,), jnp.float32),
            pltpu.VMEM((2 * l,), jnp.float32),
            pltpu.VMEM((l + 2 * _NL,), jnp.float32),
            pltpu.VMEM((nrows,), jnp.float32),
            pltpu.SemaphoreType.DMA,
            pltpu.SemaphoreType.DMA,
        ],
    )
    return f(latents)


def _encode_body(a_ref, w_ref, o_ref, *, bm):
    a = a_ref[pl.ds(pl.program_id(1) * bm, bm), :] if bm else a_ref[...]
    o_ref[...] = jax.lax.dot_general(
        a, w_ref[...],
        dimension_numbers=(((1,), (1,)), ((), ())),
        preferred_element_type=jnp.float32,
    )


def _encode(activations, w_enc, bm, bn):
    m, d = activations.shape
    l = w_enc.shape[0]
    a_whole = m * d * 4 <= 33_554_432
    if a_whole:
        bn = min(bn, 1024)
    grid = (l // bn, m // bm)
    return pl.pallas_call(
        functools.partial(_encode_body, bm=bm if a_whole else 0),
        grid=grid,
        in_specs=[
            pl.BlockSpec((m, d), lambda n, mi: (0, 0)) if a_whole
            else pl.BlockSpec((bm, d), lambda n, mi: (mi, 0)),
            pl.BlockSpec((bn, d), lambda n, mi: (n, 0)),
        ],
        out_specs=pl.BlockSpec((bm, bn), lambda n, mi: (mi, n)),
        out_shape=jax.ShapeDtypeStruct((m, l), jnp.float32),
        compiler_params=pltpu.CompilerParams(
            dimension_semantics=("arbitrary", "arbitrary"),
        ),
    )(activations, w_enc)


def _threshold_body(x_ref, t_ref, *, k):
    x = x_ref[...]
    rowmax = jnp.max(x, axis=1, keepdims=True)
    hi0 = jnp.where(
        rowmax > 0.0,
        pltpu.bitcast(rowmax, jnp.int32) + 1,
        jnp.ones_like(rowmax, jnp.int32),
    )
    lo0 = jnp.zeros_like(hi0)

    def step(_, carry):
        lo, hi = carry
        mid = lo + jax.lax.shift_right_logical(hi - lo, 1)
        mid_f = pltpu.bitcast(mid, jnp.float32)
        cnt = jnp.sum((x >= mid_f).astype(jnp.float32), axis=1, keepdims=True)
        take = cnt >= float(k)
        return jnp.where(take, mid, lo), jnp.where(take, hi, mid)

    lo, _ = jax.lax.fori_loop(0, 31, step, (lo0, hi0))
    t_ref[...] = jnp.broadcast_to(pltpu.bitcast(lo, jnp.float32), t_ref.shape)


def _thresholds(latents, br, k):
    m, l = latents.shape
    return pl.pallas_call(
        functools.partial(_threshold_body, k=k),
        grid=(m // br,),
        in_specs=[pl.BlockSpec((br, l), lambda i: (i, 0))],
        out_specs=pl.BlockSpec((br, 128), lambda i: (i, 0)),
        out_shape=jax.ShapeDtypeStruct((m, 128), jnp.float32),
        compiler_params=pltpu.CompilerParams(
            dimension_semantics=("arbitrary",),
        ),
    )(latents)


def _decode_body(x_ref, t_ref, w_ref, o_ref):
    li = pl.program_id(1)
    x = x_ref[...]
    t = t_ref[:, :1]
    s = jnp.where(x >= t, x, 0.0)
    s = jnp.maximum(s, 0.0)
    acc = jax.lax.dot_general(
        s, w_ref[...],
        dimension_numbers=(((1,), (1,)), ((), ())),
        preferred_element_type=jnp.float32,
    )

    @pl.when(li == 0)
    def _():
        o_ref[...] = acc

    @pl.when(li != 0)
    def _():
        o_ref[...] += acc


def _decode(latents, thr, w_dec, bm, bl):
    m, l = latents.shape
    d = w_dec.shape[0]
    grid = (m // bm, l // bl)
    return pl.pallas_call(
        _decode_body,
        grid=grid,
        in_specs=[
            pl.BlockSpec((bm, bl), lambda mi, li: (mi, li)),
            pl.BlockSpec((bm, 128), lambda mi, li: (mi, 0)),
            pl.BlockSpec((d, bl), lambda mi, li: (0, li)),
        ],
        out_specs=pl.BlockSpec((bm, d), lambda mi, li: (mi, 0)),
        out_shape=jax.ShapeDtypeStruct((m, d), jnp.float32),
        compiler_params=pltpu.CompilerParams(
            dimension_semantics=("arbitrary", "arbitrary"),
        ),
    )(latents, thr, w_dec)


def kernel(activations, W_enc, W_dec):
    m = activations.shape[0]
    bm_e = min(512, m)
    bn_e = min(2048, W_enc.shape[0])
    latents = _encode(activations, W_enc, bm_e, bn_e)
    if m % (_NC * _NS * _NL) == 0:
        thr1d = _thresholds_sc(latents, K_SPARSE)
        thr = jnp.broadcast_to(thr1d[:, None], (m, 128))
    else:
        thr = _thresholds(latents, min(16, m), K_SPARSE)
    bm_d = min(1024, m)
    bl_d = min(1024, W_enc.shape[0])
    return _decode(latents, thr, W_dec, bm_d, bl_d)


# speedup vs baseline: 1.0249x; 1.0249x over previous
import functools

import jax
import jax.numpy as jnp
from jax import lax
from jax.experimental import pallas as pl
from jax.experimental.pallas import tpu as pltpu
from jax.experimental.pallas import tpu_sc as plsc

K_SPARSE = 64
_NC, _NS, _NL = 2, 16, 16


def _sc_threshold_body(lat_hbm, out_hbm, row0_v, row1_v, cand_v, thr_v,
                       sem0, sem1, *, nrows, l, k):
    wid = lax.axis_index("s") * _NC + lax.axis_index("c")
    base = wid * nrows
    lanes = lax.iota(jnp.int32, _NL)
    nview = l // 128
    unroll = 8
    hi_key = jnp.full((_NL,), jnp.int32(0x7F800001))

    def process(row_v, rbase, r, acc):
        def cmax_step(i, carry):
            out = list(carry)
            for i2 in range(2):
                for j in range(8):
                    x = row_v[pl.ds(rbase + (i * 2 + i2) * 128 + j * _NL,
                                    _NL)]
                    out[j] = jnp.maximum(out[j], x)
            return tuple(out)

        zero = jnp.zeros((_NL,), jnp.float32)
        M = lax.fori_loop(0, nview // 2, cmax_step, (zero,) * 8)

        def psearch(_, lh):
            lo, hi = lh
            mid = lo + lax.shift_right_logical(hi - lo, 1)
            midf = lax.bitcast_convert_type(mid, jnp.float32)
            cnt = jnp.zeros((_NL,), jnp.int32)
            for j in range(8):
                cnt = cnt + plsc.all_reduce_population_count(M[j] >= midf)
            take = cnt >= k
            return (jnp.where(take, mid, lo), jnp.where(take, hi, mid))

        lo0 = jnp.zeros((_NL,), jnp.int32)
        pivk, _ = lax.fori_loop(0, 18, psearch, (lo0, hi_key))
        pivf = lax.bitcast_convert_type(pivk, jnp.float32)

        def comp_step(i, offs):
            xs, ms, ds = [], [], []
            for j in range(unroll):
                x = row_v[pl.ds(rbase + (i * unroll + j) * _NL, _NL)]
                m = x >= pivf
                xs.append(x)
                ms.append(m)
                ds.append(jnp.where(m, _NL, 0))
            for j in range(unroll):
                plsc.store_scatter(cand_v, [offs], xs[j], mask=ms[j])
                offs = offs + ds[j]
            return offs

        offs = lax.fori_loop(0, l // (_NL * unroll), comp_step, lanes)
        offv = lax.shift_right_logical(offs - lanes, 4)
        nsteps = jnp.max(offv)

        def csearch(_, lh):
            lo, hi = lh
            mid = lo + lax.shift_right_logical(hi - lo, 1)
            midf = lax.bitcast_convert_type(mid, jnp.float32)

            def cnt_step(s2, cv):
                for j in range(2):
                    s = s2 * 2 + j
                    xc = cand_v[pl.ds(s * _NL, _NL)]
                    msk = (xc >= midf) & (offv > s)
                    cv = cv + plsc.all_reduce_population_count(msk)
                return cv

            cv = lax.fori_loop(0, (nsteps + 1) >> 1, cnt_step,
                               jnp.zeros((_NL,), jnp.int32))
            take = cv >= k
            return (jnp.where(take, mid, lo), jnp.where(take, hi, mid))

        tk, _ = lax.fori_loop(0, 31, csearch, (pivk, hi_key))
        tf = lax.bitcast_convert_type(tk, jnp.float32)

        acc = jnp.where(lanes == (r & (_NL - 1)), tf, acc)

        @pl.when((r & (_NL - 1)) == (_NL - 1))
        def _():
            thr_v[pl.ds((r >> 4) * _NL, _NL)] = acc

        return acc

    def copy_half(r, row_v, half, sem):
        return pltpu.make_async_copy(
            lat_hbm.at[base + r], row_v.at[pl.ds(half * l, l)], sem)

    def start_pair(r, row_v, sem):
        copy_half(r, row_v, 0, sem).start()
        copy_half(r + 1, row_v, 1, sem).start()

    def wait_pair(r, row_v, sem):
        copy_half(r, row_v, 0, sem).wait()
        copy_half(r + 1, row_v, 1, sem).wait()

    start_pair(0, row0_v, sem0)

    def quad_step(q, acc):
        r = q * 4
        wait_pair(r, row0_v, sem0)
        start_pair(r + 2, row1_v, sem1)
        acc = process(row0_v, 0, r, acc)
        acc = process(row0_v, l, r + 1, acc)
        wait_pair(r + 2, row1_v, sem1)

        @pl.when(r + 4 < nrows)
        def _():
            start_pair(r + 4, row0_v, sem0)

        acc = process(row1_v, 0, r + 2, acc)
        return process(row1_v, l, r + 3, acc)

    lax.fori_loop(0, nrows // 4, quad_step, jnp.zeros((_NL,), jnp.float32))
    pltpu.sync_copy(thr_v, out_hbm.at[pl.ds(base, nrows)])


def _thresholds_sc(latents, k):
    m, l = latents.shape
    nrows = m // (_NC * _NS)
    import functools as _ft
    f = pl.kernel(
        _ft.partial(_sc_threshold_body, nrows=nrows, l=l, k=k),
        out_type=jax.ShapeDtypeStruct((m,), jnp.float32),
        mesh=plsc.VectorSubcoreMesh(core_axis_name="c", subcore_axis_name="s"),
        compiler_params=pltpu.CompilerParams(needs_layout_passes=False),
        scratch_types=[
            pltpu.VMEM((2 * l,), jnp.float32),
            pltpu.VMEM((2 * l,), jnp.float32),
            pltpu.VMEM((l + 2 * _NL,), jnp.float32),
            pltpu.VMEM((nrows,), jnp.float32),
            pltpu.SemaphoreType.DMA,
            pltpu.SemaphoreType.DMA,
        ],
    )
    return f(latents)


def _encode_body(a_ref, w_ref, o_ref):
    o_ref[...] = jax.lax.dot_general(
        a_ref[...], w_ref[...],
        dimension_numbers=(((1,), (1,)), ((), ())),
        preferred_element_type=jnp.float32,
    )


def _encode(activations, w_enc, bm, bn):
    m, d = activations.shape
    l = w_enc.shape[0]
    grid = (l // bn, m // bm)
    return pl.pallas_call(
        _encode_body,
        grid=grid,
        in_specs=[
            pl.BlockSpec((bm, d), lambda n, mi: (mi, 0)),
            pl.BlockSpec((bn, d), lambda n, mi: (n, 0)),
        ],
        out_specs=pl.BlockSpec((bm, bn), lambda n, mi: (mi, n)),
        out_shape=jax.ShapeDtypeStruct((m, l), jnp.float32),
        compiler_params=pltpu.CompilerParams(
            dimension_semantics=("arbitrary", "arbitrary"),
        ),
    )(activations, w_enc)


def _threshold_body(x_ref, t_ref, *, k):
    x = x_ref[...]
    rowmax = jnp.max(x, axis=1, keepdims=True)
    hi0 = jnp.where(
        rowmax > 0.0,
        pltpu.bitcast(rowmax, jnp.int32) + 1,
        jnp.ones_like(rowmax, jnp.int32),
    )
    lo0 = jnp.zeros_like(hi0)

    def step(_, carry):
        lo, hi = carry
        mid = lo + jax.lax.shift_right_logical(hi - lo, 1)
        mid_f = pltpu.bitcast(mid, jnp.float32)
        cnt = jnp.sum((x >= mid_f).astype(jnp.float32), axis=1, keepdims=True)
        take = cnt >= float(k)
        return jnp.where(take, mid, lo), jnp.where(take, hi, mid)

    lo, _ = jax.lax.fori_loop(0, 31, step, (lo0, hi0))
    t_ref[...] = jnp.broadcast_to(pltpu.bitcast(lo, jnp.float32), t_ref.shape)


def _thresholds(latents, br, k):
    m, l = latents.shape
    return pl.pallas_call(
        functools.partial(_threshold_body, k=k),
        grid=(m // br,),
        in_specs=[pl.BlockSpec((br, l), lambda i: (i, 0))],
        out_specs=pl.BlockSpec((br, 128), lambda i: (i, 0)),
        out_shape=jax.ShapeDtypeStruct((m, 128), jnp.float32),
        compiler_params=pltpu.CompilerParams(
            dimension_semantics=("arbitrary",),
        ),
    )(latents)


def _decode_body(x_ref, t_ref, w_ref, o_ref):
    li = pl.program_id(1)
    x = x_ref[...]
    t = t_ref[:, :1]
    s = jnp.where(x >= t, x, 0.0)
    s = jnp.maximum(s, 0.0)
    acc = jax.lax.dot_general(
        s, w_ref[...],
        dimension_numbers=(((1,), (1,)), ((), ())),
        preferred_element_type=jnp.float32,
    )

    @pl.when(li == 0)
    def _():
        o_ref[...] = acc

    @pl.when(li != 0)
    def _():
        o_ref[...] += acc


def _decode(latents, thr, w_dec, bm, bl):
    m, l = latents.shape
    d = w_dec.shape[0]
    grid = (m // bm, l // bl)
    return pl.pallas_call(
        _decode_body,
        grid=grid,
        in_specs=[
            pl.BlockSpec((bm, bl), lambda mi, li: (mi, li)),
            pl.BlockSpec((bm, 128), lambda mi, li: (mi, 0)),
            pl.BlockSpec((d, bl), lambda mi, li: (0, li)),
        ],
        out_specs=pl.BlockSpec((bm, d), lambda mi, li: (mi, 0)),
        out_shape=jax.ShapeDtypeStruct((m, d), jnp.float32),
        compiler_params=pltpu.CompilerParams(
            dimension_semantics=("arbitrary", "arbitrary"),
        ),
    )(latents, thr, w_dec)


def kernel(activations, W_enc, W_dec):
    m = activations.shape[0]
    bm_e = min(512, m)
    bn_e = min(2048, W_enc.shape[0])
    latents = _encode(activations, W_enc, bm_e, bn_e)
    if m % (_NC * _NS * _NL) == 0:
        thr1d = _thresholds_sc(latents, K_SPARSE)
        thr = jnp.broadcast_to(thr1d[:, None], (m, 128))
    else:
        thr = _thresholds(latents, min(16, m), K_SPARSE)
    bm_d = min(1024, m)
    bl_d = min(1024, W_enc.shape[0])
    return _decode(latents, thr, W_dec, bm_d, bl_d)


# speedup vs baseline: 1.0251x; 1.0002x over previous
import functools

import jax
import jax.numpy as jnp
from jax import lax
from jax.experimental import pallas as pl
from jax.experimental.pallas import tpu as pltpu
from jax.experimental.pallas import tpu_sc as plsc

K_SPARSE = 64
_NC, _NS, _NL = 2, 16, 16


def _sc_threshold_body(lat_hbm, out_hbm, row0_v, row1_v, cand_v, thr_v,
                       sem0, sem1, *, nrows, l, k):
    wid = lax.axis_index("s") * _NC + lax.axis_index("c")
    base = wid * nrows
    lanes = lax.iota(jnp.int32, _NL)
    nview = l // 128
    unroll = 8
    hi_key = jnp.full((_NL,), jnp.int32(0x7F800001))

    def process(row_v, rbase, r, acc):
        def cmax_step(i, carry):
            out = list(carry)
            for i2 in range(2):
                for j in range(8):
                    x = row_v[pl.ds(rbase + (i * 2 + i2) * 128 + j * _NL,
                                    _NL)]
                    out[j] = jnp.maximum(out[j], x)
            return tuple(out)

        zero = jnp.zeros((_NL,), jnp.float32)
        M = lax.fori_loop(0, nview // 2, cmax_step, (zero,) * 8)

        def psearch(_, lh):
            lo, hi = lh
            mid = lo + lax.shift_right_logical(hi - lo, 1)
            midf = lax.bitcast_convert_type(mid, jnp.float32)
            cnt = jnp.zeros((_NL,), jnp.int32)
            for j in range(8):
                cnt = cnt + plsc.all_reduce_population_count(M[j] >= midf)
            take = cnt >= k
            return (jnp.where(take, mid, lo), jnp.where(take, hi, mid))

        lo0 = jnp.zeros((_NL,), jnp.int32)
        pivk, _ = lax.fori_loop(0, 18, psearch, (lo0, hi_key))
        pivf = lax.bitcast_convert_type(pivk, jnp.float32)

        def comp_step(i, offs):
            xs, ms, ds = [], [], []
            for j in range(unroll):
                x = row_v[pl.ds(rbase + (i * unroll + j) * _NL, _NL)]
                m = x >= pivf
                xs.append(x)
                ms.append(m)
                ds.append(jnp.where(m, _NL, 0))
            for j in range(unroll):
                plsc.store_scatter(cand_v, [offs], xs[j], mask=ms[j])
                offs = offs + ds[j]
            return offs

        offs = lax.fori_loop(0, l // (_NL * unroll), comp_step, lanes)
        offv = lax.shift_right_logical(offs - lanes, 4)
        nsteps = jnp.max(offv)

        def csearch(_, lh):
            lo, hi = lh
            mid = lo + lax.shift_right_logical(hi - lo, 1)
            midf = lax.bitcast_convert_type(mid, jnp.float32)

            def cnt_step(s2, cv):
                for j in range(2):
                    s = s2 * 2 + j
                    xc = cand_v[pl.ds(s * _NL, _NL)]
                    msk = (xc >= midf) & (offv > s)
                    cv = cv + plsc.all_reduce_population_count(msk)
                return cv

            cv = lax.fori_loop(0, (nsteps + 1) >> 1, cnt_step,
                               jnp.zeros((_NL,), jnp.int32))
            take = cv >= k
            return (jnp.where(take, mid, lo), jnp.where(take, hi, mid))

        tk, _ = lax.fori_loop(0, 31, csearch, (pivk, hi_key))
        tf = lax.bitcast_convert_type(tk, jnp.float32)

        acc = jnp.where(lanes == (r & (_NL - 1)), tf, acc)

        @pl.when((r & (_NL - 1)) == (_NL - 1))
        def _():
            thr_v[pl.ds((r >> 4) * _NL, _NL)] = acc

        return acc

    def copy_half(r, row_v, half, sem):
        return pltpu.make_async_copy(
            lat_hbm.at[base + r], row_v.at[pl.ds(half * l, l)], sem)

    def start_pair(r, row_v, sem):
        copy_half(r, row_v, 0, sem).start()
        copy_half(r + 1, row_v, 1, sem).start()

    def wait_pair(r, row_v, sem):
        copy_half(r, row_v, 0, sem).wait()
        copy_half(r + 1, row_v, 1, sem).wait()

    start_pair(0, row0_v, sem0)

    def quad_step(q, acc):
        r = q * 4
        wait_pair(r, row0_v, sem0)
        start_pair(r + 2, row1_v, sem1)
        acc = process(row0_v, 0, r, acc)
        acc = process(row0_v, l, r + 1, acc)
        wait_pair(r + 2, row1_v, sem1)

        @pl.when(r + 4 < nrows)
        def _():
            start_pair(r + 4, row0_v, sem0)

        acc = process(row1_v, 0, r + 2, acc)
        return process(row1_v, l, r + 3, acc)

    lax.fori_loop(0, nrows // 4, quad_step, jnp.zeros((_NL,), jnp.float32))
    pltpu.sync_copy(thr_v, out_hbm.at[pl.ds(base, nrows)])


def _thresholds_sc(latents, k):
    m, l = latents.shape
    nrows = m // (_NC * _NS)
    f = pl.kernel(
        functools.partial(_sc_threshold_body, nrows=nrows, l=l, k=k),
        out_type=jax.ShapeDtypeStruct((m,), jnp.float32),
        mesh=plsc.VectorSubcoreMesh(core_axis_name="c", subcore_axis_name="s"),
        compiler_params=pltpu.CompilerParams(needs_layout_passes=False),
        scratch_types=[
            pltpu.VMEM((2 * l,), jnp.float32),
            pltpu.VMEM((2 * l,), jnp.float32),
            pltpu.VMEM((l + 2 * _NL,), jnp.float32),
            pltpu.VMEM((nrows,), jnp.float32),
            pltpu.SemaphoreType.DMA,
            pltpu.SemaphoreType.DMA,
        ],
    )
    return f(latents)


def _encode_body(a_ref, w_ref, o_ref):
    o_ref[...] = jax.lax.dot_general(
        a_ref[...], w_ref[...],
        dimension_numbers=(((1,), (1,)), ((), ())),
        preferred_element_type=jnp.float32,
    )


def _encode(activations, w_enc, bm, bn):
    m, d = activations.shape
    l = w_enc.shape[0]
    grid = (l // bn, m // bm)
    return pl.pallas_call(
        _encode_body,
        grid=grid,
        in_specs=[
            pl.BlockSpec((bm, d), lambda n, mi: (mi, 0)),
            pl.BlockSpec((bn, d), lambda n, mi: (n, 0)),
        ],
        out_specs=pl.BlockSpec((bm, bn), lambda n, mi: (mi, n)),
        out_shape=jax.ShapeDtypeStruct((m, l), jnp.float32),
        compiler_params=pltpu.CompilerParams(
            dimension_semantics=("arbitrary", "arbitrary"),
        ),
    )(activations, w_enc)


def _threshold_body(x_ref, t_ref, *, k):
    x = x_ref[...]
    rowmax = jnp.max(x, axis=1, keepdims=True)
    hi0 = jnp.where(
        rowmax > 0.0,
        pltpu.bitcast(rowmax, jnp.int32) + 1,
        jnp.ones_like(rowmax, jnp.int32),
    )
    lo0 = jnp.zeros_like(hi0)

    def step(_, carry):
        lo, hi = carry
        mid = lo + jax.lax.shift_right_logical(hi - lo, 1)
        mid_f = pltpu.bitcast(mid, jnp.float32)
        cnt = jnp.sum((x >= mid_f).astype(jnp.float32), axis=1, keepdims=True)
        take = cnt >= float(k)
        return jnp.where(take, mid, lo), jnp.where(take, hi, mid)

    lo, _ = jax.lax.fori_loop(0, 31, step, (lo0, hi0))
    t_ref[...] = jnp.broadcast_to(pltpu.bitcast(lo, jnp.float32), t_ref.shape)


def _thresholds(latents, br, k):
    m, l = latents.shape
    return pl.pallas_call(
        functools.partial(_threshold_body, k=k),
        grid=(m // br,),
        in_specs=[pl.BlockSpec((br, l), lambda i: (i, 0))],
        out_specs=pl.BlockSpec((br, 128), lambda i: (i, 0)),
        out_shape=jax.ShapeDtypeStruct((m, 128), jnp.float32),
        compiler_params=pltpu.CompilerParams(
            dimension_semantics=("arbitrary",),
        ),
    )(latents)


def _decode_body(x_ref, t_ref, w_ref, o_ref):
    li = pl.program_id(1)
    x = x_ref[...]
    t = t_ref[:, :1]
    s = jnp.where(x >= t, x, 0.0)
    s = jnp.maximum(s, 0.0)
    acc = jax.lax.dot_general(
        s, w_ref[...],
        dimension_numbers=(((1,), (1,)), ((), ())),
        preferred_element_type=jnp.float32,
    )

    @pl.when(li == 0)
    def _():
        o_ref[...] = acc

    @pl.when(li != 0)
    def _():
        o_ref[...] += acc


def _decode(latents, thr, w_dec, bm, bl):
    m, l = latents.shape
    d = w_dec.shape[0]
    grid = (m // bm, l // bl)
    return pl.pallas_call(
        _decode_body,
        grid=grid,
        in_specs=[
            pl.BlockSpec((bm, bl), lambda mi, li: (mi, li)),
            pl.BlockSpec((bm, 128), lambda mi, li: (mi, 0)),
            pl.BlockSpec((d, bl), lambda mi, li: (0, li)),
        ],
        out_specs=pl.BlockSpec((bm, d), lambda mi, li: (mi, 0)),
        out_shape=jax.ShapeDtypeStruct((m, d), jnp.float32),
        compiler_params=pltpu.CompilerParams(
            dimension_semantics=("arbitrary", "arbitrary"),
        ),
    )(latents, thr, w_dec)


def kernel(activations, W_enc, W_dec):
    m = activations.shape[0]
    bm_e = min(512, m)
    bn_e = min(2048, W_enc.shape[0])
    latents = _encode(activations, W_enc, bm_e, bn_e)
    if m % (_NC * _NS * _NL) == 0:
        thr1d = _thresholds_sc(latents, K_SPARSE)
        thr = jnp.broadcast_to(thr1d[:, None], (m, 128))
    else:
        thr = _thresholds(latents, min(16, m), K_SPARSE)
    bm_d = min(1024, m)
    bl_d = min(1024, W_enc.shape[0])
    return _decode(latents, thr, W_dec, bm_d, bl_d)
